# baseline (speedup 1.0000x reference)
import functools

import jax
import jax.numpy as jnp
from jax import lax
from jax.experimental import pallas as pl
from jax.experimental.pallas import tpu as pltpu
from jax.experimental.pallas import tpu_sc as plsc

BATCH = 16384
D = 16
NB = BATCH // 128
NW = 32
ROWS_PER_W = NB // NW


def _make_gather():
    mesh = plsc.VectorSubcoreMesh(core_axis_name="c", subcore_axis_name="s")
    out3 = jax.ShapeDtypeStruct((NB, 128, D), jnp.float32)

    @functools.partial(
        pl.kernel,
        mesh=mesh,
        out_type=[out3, out3, out3],
        compiler_params=pltpu.CompilerParams(use_tc_tiling_on_sc=False),
        scratch_types=[
            pltpu.VMEM((ROWS_PER_W, 128), jnp.int32),
            pltpu.VMEM((ROWS_PER_W, 128), jnp.int32),
            pltpu.VMEM((ROWS_PER_W, 128), jnp.int32),
            pltpu.VMEM((ROWS_PER_W, 128, D), jnp.float32),
            pltpu.VMEM((ROWS_PER_W, 128, D), jnp.float32),
            pltpu.VMEM((ROWS_PER_W, 128, D), jnp.float32),
            pltpu.SemaphoreType.DMA,
        ],
    )
    def gather(u_idx_hbm, i_idx_hbm, r_idx_hbm, ut_hbm, it_hbm, rt_hbm,
               out_u, out_i, out_r,
               uix, iix, rix, urow, irow, rrow, sem):
        wid = lax.axis_index("s") * 2 + lax.axis_index("c")
        base = wid * ROWS_PER_W
        pltpu.sync_copy(u_idx_hbm.at[pl.ds(base, ROWS_PER_W), :], uix)
        pltpu.sync_copy(i_idx_hbm.at[pl.ds(base, ROWS_PER_W), :], iix)
        pltpu.sync_copy(r_idx_hbm.at[pl.ds(base, ROWS_PER_W), :], rix)
        copies = []
        for c in range(ROWS_PER_W):
            copies.append(pltpu.async_copy(ut_hbm.at[uix.at[c]], urow.at[c], sem))
            copies.append(pltpu.async_copy(it_hbm.at[iix.at[c]], irow.at[c], sem))
            copies.append(pltpu.async_copy(rt_hbm.at[rix.at[c]], rrow.at[c], sem))
        for cp in copies:
            cp.wait()
        pltpu.sync_copy(urow, out_u.at[pl.ds(base, ROWS_PER_W)])
        pltpu.sync_copy(irow, out_i.at[pl.ds(base, ROWS_PER_W)])
        pltpu.sync_copy(rrow, out_r.at[pl.ds(base, ROWS_PER_W)])

    return gather


_gather = _make_gather()

_BM = 2048


def _mlp_body(u_ref, i_ref, r_ref, w1_ref, b1_ref, w2_ref, b2_ref, out_ref):
    w1 = w1_ref[...]
    dn = (((1,), (1,)), ((), ()))
    h = lax.dot_general(u_ref[...], w1[:, 0:16], dn,
                        preferred_element_type=jnp.float32)
    h += lax.dot_general(i_ref[...], w1[:, 16:32], dn,
                         preferred_element_type=jnp.float32)
    h += lax.dot_general(r_ref[...], w1[:, 32:48], dn,
                         preferred_element_type=jnp.float32)
    h += b1_ref[...]
    h = jnp.maximum(h, 0.0)
    p = jnp.sum(h * w2_ref[...], axis=1, keepdims=True)
    p += b2_ref[0, 0]
    out_ref[...] = 10.0 / (1.0 + jnp.exp(-p))


@jax.jit
def _mlp(emb_u, emb_i, emb_r, W1, b1, W2, b2):
    grid = (BATCH // _BM,)
    return pl.pallas_call(
        _mlp_body,
        grid=grid,
        in_specs=[
            pl.BlockSpec((_BM, D), lambda i: (i, 0)),
            pl.BlockSpec((_BM, D), lambda i: (i, 0)),
            pl.BlockSpec((_BM, D), lambda i: (i, 0)),
            pl.BlockSpec((128, 48), lambda i: (0, 0)),
            pl.BlockSpec((1, 128), lambda i: (0, 0)),
            pl.BlockSpec((1, 128), lambda i: (0, 0)),
            pl.BlockSpec((1, 1), lambda i: (0, 0)),
        ],
        out_specs=pl.BlockSpec((_BM, 1), lambda i: (i, 0)),
        out_shape=jax.ShapeDtypeStruct((BATCH, 1), jnp.float32),
    )(emb_u, emb_i, emb_r, W1, b1, W2, b2)


def kernel(X, user_table, item_table, rating_table, W1, b1, W2, b2):
    Xi = X.astype(jnp.int32)
    u_idx = Xi[:, 0].reshape(NB, 128)
    i_idx = Xi[:, 1].reshape(NB, 128)
    r_idx = Xi[:, 2].reshape(NB, 128)
    eu, ei, er = _gather(u_idx, i_idx, r_idx, user_table, item_table,
                         rating_table)
    emb_u = eu.reshape(BATCH, D)
    emb_i = ei.reshape(BATCH, D)
    emb_r = er.reshape(BATCH, D)
    return _mlp(emb_u, emb_i, emb_r, W1,
                b1.reshape(1, 128), W2, b2.reshape(1, 1))
